# Initial kernel scaffold; baseline (speedup 1.0000x reference)
import jax
import jax.numpy as jnp
from jax.experimental import pallas as pl


def kernel(edge_index, h, W, a_l, a_r):
    raise NotImplementedError("write your pallas kernel here")


# speedup vs baseline: 36.4465x; 36.4465x over previous
import jax
import jax.numpy as jnp
from jax import lax
from jax.experimental import pallas as pl
from jax.experimental.pallas import tpu as pltpu
from jax.experimental.pallas import tpu_sc as plsc

N = 10000
E = 320000
IN_SIZE = 128
OUT_SIZE = 32
N_HEADS = 4
C = OUT_SIZE * N_HEADS
CH = C // 2
SW = 16

NC = 2
NS = 16
EDGES_PER_TILE = E // NS
BLK = 80
NBLK = EDGES_PER_TILE // BLK
ROWS_PER_TILE = 624
ROWS_TAIL = N - NS * ROWS_PER_TILE
LANES = 16


def _proj_body(h_ref, w_ref, al_ref, ar_ref, gl_ref, gr_ref,
               wh_ref, wh12_ref):
    wh = jnp.dot(h_ref[...], w_ref[...], preferred_element_type=jnp.float32)
    wh_ref[0] = wh[:, :CH]
    wh_ref[1] = wh[:, CH:]
    tl = wh * al_ref[...]
    tr = wh * ar_ref[...]
    w12 = (
        jnp.dot(tl, gl_ref[...], preferred_element_type=jnp.float32,
                precision=lax.Precision.HIGHEST)
        + jnp.dot(tr, gr_ref[...], preferred_element_type=jnp.float32,
                  precision=lax.Precision.HIGHEST)
    )
    wh12_ref[0] = w12
    wh12_ref[1] = w12


def _projections(h, W, a_l, a_r):
    bn = 2000
    grid = N // bn
    al_row = a_l.reshape(1, C)
    ar_row = a_r.reshape(1, C)
    heads = jnp.arange(C, dtype=jnp.int32) // OUT_SIZE
    cols = jnp.arange(SW, dtype=jnp.int32)
    gl = (heads[:, None] == cols[None, :]).astype(jnp.float32)
    gr = (heads[:, None] == (cols[None, :] - 4)).astype(jnp.float32)
    return pl.pallas_call(
        _proj_body,
        grid=(grid,),
        in_specs=[
            pl.BlockSpec((bn, IN_SIZE), lambda i: (i, 0)),
            pl.BlockSpec((IN_SIZE, C), lambda i: (0, 0)),
            pl.BlockSpec((1, C), lambda i: (0, 0)),
            pl.BlockSpec((1, C), lambda i: (0, 0)),
            pl.BlockSpec((C, SW), lambda i: (0, 0)),
            pl.BlockSpec((C, SW), lambda i: (0, 0)),
        ],
        out_specs=[
            pl.BlockSpec((NC, bn, CH), lambda i: (0, i, 0)),
            pl.BlockSpec((NC, bn, SW), lambda i: (0, i, 0)),
        ],
        out_shape=[
            jax.ShapeDtypeStruct((NC, N, CH), jnp.float32),
            jax.ShapeDtypeStruct((NC, N, SW), jnp.float32),
        ],
    )(h, W, al_row, ar_row, gl, gr)


def _edge_body(row_hbm, colb_hbm, whcat_ref, wh12cat_ref,
               u0_out, u1_out, s_out,
               row_l, col_l, w1buf, w2buf, whbuf, exbuf, u_sh, s_sh,
               sem, sem2):
    cid = lax.axis_index("c")
    sid = lax.axis_index("s")

    iota = lax.iota(jnp.int32, LANES)
    lane_mod4 = iota % 4
    zf = jnp.zeros((LANES,), jnp.float32)

    for e in range(BLK):
        for k in range(CH // LANES):
            whbuf[e, pl.ds(k * LANES, LANES)] = zf
    for e in range(BLK):
        plsc.store_scatter(exbuf, [jnp.full((LANES,), e, jnp.int32), iota], zf)

    base_row = sid * ROWS_PER_TILE
    nfull = ROWS_PER_TILE // BLK
    rem = ROWS_PER_TILE - nfull * BLK
    for t in range(nfull):
        pltpu.sync_copy(whbuf, u_sh.at[pl.ds(base_row + t * BLK, BLK)])
        pltpu.sync_copy(exbuf, s_sh.at[pl.ds(base_row + t * BLK, BLK)])
    pltpu.sync_copy(whbuf.at[pl.ds(0, rem)],
                    u_sh.at[pl.ds(base_row + nfull * BLK, rem)])
    pltpu.sync_copy(exbuf.at[pl.ds(0, rem)],
                    s_sh.at[pl.ds(base_row + nfull * BLK, rem)])

    @pl.when(sid == 0)
    def _zero_tail():
        pltpu.sync_copy(whbuf.at[pl.ds(0, ROWS_TAIL)],
                        u_sh.at[pl.ds(NS * ROWS_PER_TILE, ROWS_TAIL)])
        pltpu.sync_copy(exbuf.at[pl.ds(0, ROWS_TAIL)],
                        s_sh.at[pl.ds(NS * ROWS_PER_TILE, ROWS_TAIL)])

    pltpu.sync_copy(row_hbm.at[sid], row_l)
    pltpu.sync_copy(colb_hbm.at[cid, sid], col_l)

    plsc.subcore_barrier()

    def block(j, carry):
        cp = pltpu.async_copy(whcat_ref.at[col_l.at[j]], whbuf, sem)
        pltpu.async_copy(wh12cat_ref.at[row_l.at[j]], w1buf, sem2).wait()
        pltpu.async_copy(wh12cat_ref.at[col_l.at[j]], w2buf, sem2).wait()
        cp.wait()

        for g in range(BLK // LANES):
            pos = jnp.full((LANES,), g * LANES, jnp.int32) + iota
            for h in range(N_HEADS):
                hv = jnp.full((LANES,), h, jnp.int32)
                w1 = plsc.load_gather(w1buf, [pos, hv])
                w2 = plsc.load_gather(w2buf, [pos, hv + 4])
                e = w1 + w2
                e = jnp.where(e > 0, e, 0.2 * e)
                ex = jnp.exp(e)
                plsc.store_scatter(exbuf, [pos, hv], ex)

        for e in range(BLK):
            exrep = plsc.load_gather(
                exbuf, [jnp.full((LANES,), e, jnp.int32), lane_mod4])
            for k in range(CH // LANES):
                sl = pl.ds(k * LANES, LANES)
                whbuf[e, sl] = whbuf[e, sl] * exrep

        pltpu.sync_copy(exbuf, s_sh.at[row_l.at[j]], add=True)
        pltpu.sync_copy(whbuf, u_sh.at[row_l.at[j]], add=True)
        return carry

    lax.fori_loop(0, NBLK, block, 0)

    plsc.subcore_barrier()

    @pl.when(cid == 0)
    def _emit0():
        pltpu.sync_copy(u_sh.at[pl.ds(base_row, ROWS_PER_TILE)],
                        u0_out.at[pl.ds(base_row, ROWS_PER_TILE)])
        pltpu.sync_copy(s_sh.at[pl.ds(base_row, ROWS_PER_TILE)],
                        s_out.at[pl.ds(base_row, ROWS_PER_TILE)])

        @pl.when(sid == 0)
        def _tail0():
            pltpu.sync_copy(u_sh.at[pl.ds(NS * ROWS_PER_TILE, ROWS_TAIL)],
                            u0_out.at[pl.ds(NS * ROWS_PER_TILE, ROWS_TAIL)])
            pltpu.sync_copy(s_sh.at[pl.ds(NS * ROWS_PER_TILE, ROWS_TAIL)],
                            s_out.at[pl.ds(NS * ROWS_PER_TILE, ROWS_TAIL)])

    @pl.when(cid == 1)
    def _emit1():
        pltpu.sync_copy(u_sh.at[pl.ds(base_row, ROWS_PER_TILE)],
                        u1_out.at[pl.ds(base_row, ROWS_PER_TILE)])

        @pl.when(sid == 0)
        def _tail1():
            pltpu.sync_copy(u_sh.at[pl.ds(NS * ROWS_PER_TILE, ROWS_TAIL)],
                            u1_out.at[pl.ds(NS * ROWS_PER_TILE, ROWS_TAIL)])


def _edge_phase(row_r, colb_r, whcat, wh12cat):
    mesh = plsc.VectorSubcoreMesh(
        core_axis_name="c", subcore_axis_name="s", num_cores=NC, num_subcores=NS)
    f = pl.kernel(
        _edge_body,
        out_type=[
            jax.ShapeDtypeStruct((N, CH), jnp.float32),
            jax.ShapeDtypeStruct((N, CH), jnp.float32),
            jax.ShapeDtypeStruct((N, SW), jnp.float32),
        ],
        mesh=mesh,
        scratch_types=[
            pltpu.VMEM((NBLK, BLK), jnp.int32),
            pltpu.VMEM((NBLK, BLK), jnp.int32),
            pltpu.VMEM((BLK, SW), jnp.float32),
            pltpu.VMEM((BLK, SW), jnp.float32),
            pltpu.VMEM((BLK, CH), jnp.float32),
            pltpu.VMEM((BLK, SW), jnp.float32),
            pltpu.VMEM_SHARED((N, CH), jnp.float32),
            pltpu.VMEM_SHARED((N, SW), jnp.float32),
            pltpu.SemaphoreType.DMA,
            pltpu.SemaphoreType.DMA,
        ],
        compiler_params=pltpu.CompilerParams(
            use_tc_tiling_on_sc=False, needs_layout_passes=False),
    )
    return f(row_r, colb_r, whcat, wh12cat)


def _norm_body(u0_ref, u1_ref, s_ref, r_ref, out_ref):
    u = jnp.concatenate([u0_ref[...], u1_ref[...]], axis=1)
    srep = jnp.dot(s_ref[...], r_ref[...], preferred_element_type=jnp.float32,
                   precision=lax.Precision.HIGHEST)
    out_ref[...] = jnp.where(srep > 0, jnp.maximum(u / srep, 0.0), 0.0)


def _normalize(u0, u1, s):
    bn = 2000
    grid = N // bn
    rmat = (jnp.arange(SW, dtype=jnp.int32)[:, None]
            == (jnp.arange(C, dtype=jnp.int32)[None, :] % N_HEADS)).astype(jnp.float32)
    return pl.pallas_call(
        _norm_body,
        grid=(grid,),
        in_specs=[
            pl.BlockSpec((bn, CH), lambda i: (i, 0)),
            pl.BlockSpec((bn, CH), lambda i: (i, 0)),
            pl.BlockSpec((bn, SW), lambda i: (i, 0)),
            pl.BlockSpec((SW, C), lambda i: (0, 0)),
        ],
        out_specs=pl.BlockSpec((bn, C), lambda i: (i, 0)),
        out_shape=jax.ShapeDtypeStruct((N, C), jnp.float32),
    )(u0, u1, s, rmat)


def kernel(edge_index, h, W, a_l, a_r):
    wh_split, wh12d = _projections(h, W, a_l, a_r)
    whcat = wh_split.reshape(NC * N, CH)
    wh12cat = wh12d.reshape(NC * N, SW)
    row_r = edge_index[0].reshape(NS, NBLK, BLK)
    col_r = edge_index[1].reshape(NS, NBLK, BLK)
    colb_r = jnp.stack([col_r, col_r + N])
    u0, u1, s = _edge_phase(row_r, colb_r, whcat, wh12cat)
    out = _normalize(u0, u1, s)
    return out.reshape(N, OUT_SIZE, N_HEADS)
